# baseline (speedup 1.0000x reference)
import functools

import jax
import jax.numpy as jnp
from jax import lax
from jax.experimental import pallas as pl
from jax.experimental.pallas import tpu as pltpu


def _gcn_kernel(x_ref, wt_ref, b_ref, adj_hbm, out_hbm,
                h_ref, ab0, ab1, ob0, ob1, sa0, sa1, so0, so1,
                *, tm, nb):
    abufs = (ab0, ab1)
    obufs = (ob0, ob1)
    asems = (sa0, sa1)
    osems = (so0, so1)

    def adj_copy(i):
        return pltpu.make_async_copy(
            adj_hbm.at[pl.ds(i * tm, tm)], abufs[i % 2], asems[i % 2])

    def out_copy(i):
        return pltpu.make_async_copy(
            obufs[i % 2], out_hbm.at[pl.ds(i * tm, tm)], osems[i % 2])

    adj_copy(0).start()
    if nb > 1:
        adj_copy(1).start()

    h_ref[...] = jnp.dot(x_ref[...], wt_ref[...],
                         preferred_element_type=jnp.float32,
                         precision=lax.Precision.DEFAULT) + b_ref[...]

    for i in range(nb):
        adj_copy(i).wait()
        g = jnp.dot(abufs[i % 2][...], h_ref[...],
                    preferred_element_type=jnp.float32,
                    precision=lax.Precision.DEFAULT)
        if i >= 2:
            out_copy(i - 2).wait()
        obufs[i % 2][...] = jnp.maximum(g, 0.0)
        out_copy(i).start()
        if i + 2 < nb:
            adj_copy(i + 2).start()

    for i in range(max(nb - 2, 0), nb):
        out_copy(i).wait()


def _pick_tile(m, candidates):
    for c in candidates:
        if m % c == 0:
            return c
    return m


def kernel(x, adj, W, b):
    n_nodes, d_in = x.shape
    d_out = W.shape[0]
    m_rows = adj.shape[0]

    wt = W.T
    b2 = b.reshape(1, d_out)

    tm = _pick_tile(m_rows, (400, 200, 8, 1))
    nb = m_rows // tm
    body = functools.partial(_gcn_kernel, tm=tm, nb=nb)
    out = pl.pallas_call(
        body,
        in_specs=[
            pl.BlockSpec(memory_space=pltpu.VMEM),
            pl.BlockSpec(memory_space=pltpu.VMEM),
            pl.BlockSpec(memory_space=pltpu.VMEM),
            pl.BlockSpec(memory_space=pl.ANY),
        ],
        out_specs=pl.BlockSpec(memory_space=pl.ANY),
        out_shape=jax.ShapeDtypeStruct((m_rows, d_out), jnp.float32),
        scratch_shapes=[
            pltpu.VMEM((n_nodes, d_out), jnp.float32),
            pltpu.VMEM((tm, n_nodes), jnp.float32),
            pltpu.VMEM((tm, n_nodes), jnp.float32),
            pltpu.VMEM((tm, d_out), jnp.float32),
            pltpu.VMEM((tm, d_out), jnp.float32),
            pltpu.SemaphoreType.DMA,
            pltpu.SemaphoreType.DMA,
            pltpu.SemaphoreType.DMA,
            pltpu.SemaphoreType.DMA,
        ],
    )(x, wt, b2, adj)
    return out


# speedup vs baseline: 1.0814x; 1.0814x over previous
import jax
import jax.numpy as jnp
from jax import lax
from jax.experimental import pallas as pl
from jax.experimental.pallas import tpu as pltpu


def _gcn_kernel(x_ref, wt_ref, b_ref, adj_ref, out_ref, h_ref):
    @pl.when(pl.program_id(0) == 0)
    def _compute_h():
        h = jnp.dot(x_ref[...], wt_ref[...],
                    preferred_element_type=jnp.float32,
                    precision=lax.Precision.DEFAULT)
        h_ref[...] = h + b_ref[...]

    acc = jnp.dot(adj_ref[...], h_ref[...],
                  preferred_element_type=jnp.float32,
                  precision=lax.Precision.DEFAULT)
    out_ref[...] = jnp.maximum(acc, 0.0)


def _pick_tile(m, candidates):
    for c in candidates:
        if m % c == 0:
            return c
    return m


def kernel(x, adj, W, b):
    n_nodes, d_in = x.shape
    d_out = W.shape[0]
    m_rows = adj.shape[0]

    wt = W.T
    b2 = b.reshape(1, d_out)

    tm = _pick_tile(m_rows, (400, 250, 200, 500, 100, 8, 1))
    out = pl.pallas_call(
        _gcn_kernel,
        grid=(m_rows // tm,),
        in_specs=[
            pl.BlockSpec((n_nodes, d_in), lambda i: (0, 0)),
            pl.BlockSpec((d_in, d_out), lambda i: (0, 0)),
            pl.BlockSpec((1, d_out), lambda i: (0, 0)),
            pl.BlockSpec((tm, n_nodes), lambda i: (i, 0)),
        ],
        out_specs=pl.BlockSpec((tm, d_out), lambda i: (i, 0)),
        out_shape=jax.ShapeDtypeStruct((m_rows, d_out), jnp.float32),
        scratch_shapes=[pltpu.VMEM((n_nodes, d_out), jnp.float32)],
    )(x, wt, b2, adj)
    return out
